# Initial kernel scaffold; baseline (speedup 1.0000x reference)
import jax
import jax.numpy as jnp
from jax.experimental import pallas as pl


def kernel(x, batch, W, b):
    raise NotImplementedError("write your pallas kernel here")


# speedup vs baseline: 1.5679x; 1.5679x over previous
import functools

import jax
import jax.numpy as jnp
from jax import lax
from jax.experimental import pallas as pl
from jax.experimental.pallas import tpu as pltpu
from jax.experimental.pallas import tpu_sc as plsc

N_NODES = 100000
D = 512
S = 1024
NC = 2
NS = 16
NW = NC * NS
RPT = N_NODES // NW
C = 25
NCHUNK = RPT // C
L = 16
DV = D // L
SEG_PER_TILE = S // NS


def _body(x_hbm, idx_hbm, w_hbm, b_hbm, out_hbm, xbuf, idxbuf, wbuf, bbuf,
          zbuf, acc):
    c = lax.axis_index("c")
    s = lax.axis_index("s")
    wid = c * NS + s
    row0 = wid * RPT

    pltpu.sync_copy(w_hbm, wbuf)
    pltpu.sync_copy(b_hbm, bbuf)
    pltpu.sync_copy(idx_hbm.at[wid], idxbuf)

    def _zero_row(r, _):
        for j in range(DV):
            zbuf[r, pl.ds(L * j, L)] = jnp.zeros((L,), jnp.float32)
        return 0
    lax.fori_loop(0, SEG_PER_TILE, _zero_row, 0)
    pltpu.sync_copy(zbuf, acc.at[pl.ds(s * SEG_PER_TILE, SEG_PER_TILE)])
    plsc.subcore_barrier()

    bias = bbuf[:]

    def _chunk(k, _):
        pltpu.sync_copy(x_hbm.at[pl.ds(row0 + k * C, C)], xbuf)

        def _row(r, _):
            xs = [xbuf[r, pl.ds(L * j, L)] for j in range(DV)]
            accv = xs[0] * wbuf[pl.ds(0, L)]
            for j in range(1, DV):
                accv = accv + xs[j] * wbuf[pl.ds(L * j, L)]
            dot = jnp.sum(accv)
            z = jnp.full((L,), dot, jnp.float32) + bias
            wv = 1.0 / (1.0 + jnp.exp(-z))
            for j in range(DV):
                xbuf[r, pl.ds(L * j, L)] = xs[j] * wv
            return 0
        lax.fori_loop(0, C, _row, 0)

        pltpu.sync_copy(xbuf, acc.at[idxbuf.at[k]], add=True)
        return 0
    lax.fori_loop(0, NCHUNK, _chunk, 0)

    plsc.subcore_barrier()
    pltpu.sync_copy(acc.at[pl.ds(s * SEG_PER_TILE, SEG_PER_TILE)],
                    out_hbm.at[pl.ds(c * S + s * SEG_PER_TILE, SEG_PER_TILE)])


@jax.jit
def _weight_and_sum(x, idx3, w_flat, b16):
    mesh = plsc.VectorSubcoreMesh(core_axis_name="c", subcore_axis_name="s",
                                  num_cores=NC, num_subcores=NS)
    f = pl.kernel(
        _body,
        out_type=jax.ShapeDtypeStruct((NC * S, D), jnp.float32),
        mesh=mesh,
        scratch_types=[
            pltpu.VMEM((C, D), jnp.float32),
            pltpu.VMEM((NCHUNK, C), jnp.int32),
            pltpu.VMEM((D,), jnp.float32),
            pltpu.VMEM((L,), jnp.float32),
            pltpu.VMEM((SEG_PER_TILE, D), jnp.float32),
            pltpu.VMEM_SHARED((S, D), jnp.float32),
        ],
        compiler_params=pltpu.CompilerParams(use_tc_tiling_on_sc=False,
                                             needs_layout_passes=False),
    )
    partials = f(x, idx3, w_flat, b16)
    return partials[:S] + partials[S:]


def kernel(x, batch, W, b):
    idx3 = batch.reshape(NW, NCHUNK, C)
    w_flat = W.reshape(D)
    b16 = jnp.broadcast_to(b, (L,))
    return _weight_and_sum(x, idx3, w_flat, b16)


# speedup vs baseline: 1.6146x; 1.0298x over previous
import functools

import jax
import jax.numpy as jnp
from jax import lax
from jax.experimental import pallas as pl
from jax.experimental.pallas import tpu as pltpu
from jax.experimental.pallas import tpu_sc as plsc

N_NODES = 100000
D = 512
S = 1024
NC = 2
NS = 16
NW = NC * NS
RPT = N_NODES // NW
C = 25
NCHUNK = RPT // C
L = 16
DV = D // L
SEG_PER_TILE = S // NS


def _body(x_hbm, idx_hbm, w_hbm, b_hbm, out_hbm, xbuf, idxbuf, wbuf, bbuf,
          zbuf, acc):
    c = lax.axis_index("c")
    s = lax.axis_index("s")
    wid = c * NS + s
    row0 = wid * RPT

    pltpu.sync_copy(w_hbm, wbuf)
    pltpu.sync_copy(b_hbm, bbuf)
    pltpu.sync_copy(idx_hbm.at[wid], idxbuf)

    def _zero_row(r, _):
        for j in range(DV):
            zbuf[r, pl.ds(L * j, L)] = jnp.zeros((L,), jnp.float32)
        return 0
    lax.fori_loop(0, SEG_PER_TILE, _zero_row, 0)
    pltpu.sync_copy(zbuf, acc.at[pl.ds(s * SEG_PER_TILE, SEG_PER_TILE)])
    plsc.subcore_barrier()

    bias = bbuf[:]

    def _do_row(r):
        xs = [xbuf[r, pl.ds(L * j, L)] for j in range(DV)]
        accs = [xs[j] * wbuf[pl.ds(L * j, L)] for j in range(4)]
        for j in range(4, DV):
            accs[j % 4] = accs[j % 4] + xs[j] * wbuf[pl.ds(L * j, L)]
        accv = (accs[0] + accs[1]) + (accs[2] + accs[3])
        dot = jnp.sum(accv)
        z = jnp.full((L,), dot, jnp.float32) + bias
        wv = 1.0 / (1.0 + jnp.exp(-z))
        for j in range(DV):
            xbuf[r, pl.ds(L * j, L)] = xs[j] * wv

    def _chunk(k, _):
        pltpu.sync_copy(x_hbm.at[pl.ds(row0 + k * C, C)], xbuf)

        def _row_pair(r, _):
            _do_row(2 * r)
            _do_row(2 * r + 1)
            return 0
        lax.fori_loop(0, C // 2, _row_pair, 0)
        _do_row(C - 1)

        pltpu.sync_copy(xbuf, acc.at[idxbuf.at[k]], add=True)
        return 0
    lax.fori_loop(0, NCHUNK, _chunk, 0)

    plsc.subcore_barrier()
    pltpu.sync_copy(acc.at[pl.ds(s * SEG_PER_TILE, SEG_PER_TILE)],
                    out_hbm.at[pl.ds(c * S + s * SEG_PER_TILE, SEG_PER_TILE)])


@jax.jit
def _weight_and_sum(x, idx3, w_flat, b16):
    mesh = plsc.VectorSubcoreMesh(core_axis_name="c", subcore_axis_name="s",
                                  num_cores=NC, num_subcores=NS)
    f = pl.kernel(
        _body,
        out_type=jax.ShapeDtypeStruct((NC * S, D), jnp.float32),
        mesh=mesh,
        scratch_types=[
            pltpu.VMEM((C, D), jnp.float32),
            pltpu.VMEM((NCHUNK, C), jnp.int32),
            pltpu.VMEM((D,), jnp.float32),
            pltpu.VMEM((L,), jnp.float32),
            pltpu.VMEM((SEG_PER_TILE, D), jnp.float32),
            pltpu.VMEM_SHARED((S, D), jnp.float32),
        ],
        compiler_params=pltpu.CompilerParams(use_tc_tiling_on_sc=False,
                                             needs_layout_passes=False),
    )
    partials = f(x, idx3, w_flat, b16)
    return partials[:S] + partials[S:]


def kernel(x, batch, W, b):
    idx3 = batch.reshape(NW, NCHUNK, C)
    w_flat = W.reshape(D)
    b16 = jnp.broadcast_to(b, (L,))
    return _weight_and_sum(x, idx3, w_flat, b16)


# speedup vs baseline: 2.3639x; 1.4640x over previous
import jax
import jax.numpy as jnp
from jax import lax
from jax.experimental import pallas as pl
from jax.experimental.pallas import tpu as pltpu
from jax.experimental.pallas import tpu_sc as plsc

N_NODES = 100000
D = 512
S = 1024
NC = 2
NS = 16
NW = NC * NS
RPT = N_NODES // NW
C = 25
NCHUNK = RPT // C
L = 16
DV = D // L
SEG_PER_TILE = S // NS


def _body(x_hbm, idx_hbm, w_hbm, b_hbm, out_hbm, xb0, xb1, xb2, idxbuf,
          wbuf, bbuf, zbuf, acc, ls0, ls1, ls2, ss0, ss1, ss2):
    c = lax.axis_index("c")
    s = lax.axis_index("s")
    wid = c * NS + s
    row0 = wid * RPT

    pltpu.sync_copy(w_hbm, wbuf)
    pltpu.sync_copy(b_hbm, bbuf)
    pltpu.sync_copy(idx_hbm.at[wid], idxbuf)

    def _zero_row(r, _):
        for j in range(DV):
            zbuf[r, pl.ds(L * j, L)] = jnp.zeros((L,), jnp.float32)
        return 0
    lax.fori_loop(0, SEG_PER_TILE, _zero_row, 0)
    pltpu.sync_copy(zbuf, acc.at[pl.ds(s * SEG_PER_TILE, SEG_PER_TILE)])
    plsc.subcore_barrier()

    bias = bbuf[:]

    def _load(k, xb, sem):
        pltpu.async_copy(x_hbm.at[pl.ds(row0 + k * C, C)], xb, sem)

    def _wait_load(xb, sem):
        pltpu.make_async_copy(x_hbm.at[pl.ds(row0, C)], xb, sem).wait()

    def _scat(k, xb, sem):
        pltpu.async_copy(xb, acc.at[idxbuf.at[k]], sem, add=True)

    def _wait_scat(xb, sem):
        pltpu.make_async_copy(xb, acc.at[idxbuf.at[0]], sem).wait()

    def _do_row(xb, r):
        xs = [xb[r, pl.ds(L * j, L)] for j in range(DV)]
        accs = [xs[j] * wbuf[pl.ds(L * j, L)] for j in range(4)]
        for j in range(4, DV):
            accs[j % 4] = accs[j % 4] + xs[j] * wbuf[pl.ds(L * j, L)]
        accv = (accs[0] + accs[1]) + (accs[2] + accs[3])
        dot = jnp.sum(accv)
        z = jnp.full((L,), dot, jnp.float32) + bias
        wv = 1.0 / (1.0 + jnp.exp(-z))
        for j in range(DV):
            xb[r, pl.ds(L * j, L)] = xs[j] * wv

    def _compute(xb):
        def _row_pair(r, _):
            _do_row(xb, 2 * r)
            _do_row(xb, 2 * r + 1)
            return 0
        lax.fori_loop(0, C // 2, _row_pair, 0)
        _do_row(xb, C - 1)

    _load(0, xb0, ls0)
    _load(1, xb1, ls1)

    _wait_load(xb0, ls0)
    _compute(xb0)
    _scat(0, xb0, ss0)
    _load(2, xb2, ls2)

    _wait_load(xb1, ls1)
    _compute(xb1)
    _scat(1, xb1, ss1)
    _wait_scat(xb0, ss0)
    _load(3, xb0, ls0)

    _wait_load(xb2, ls2)
    _compute(xb2)
    _scat(2, xb2, ss2)
    _wait_scat(xb1, ss1)
    _load(4, xb1, ls1)

    def _triple(i, _):
        k = 3 * i
        _wait_load(xb0, ls0)
        _compute(xb0)
        _scat(k, xb0, ss0)
        _wait_scat(xb2, ss2)
        _load(k + 2, xb2, ls2)

        _wait_load(xb1, ls1)
        _compute(xb1)
        _scat(k + 1, xb1, ss1)
        _wait_scat(xb0, ss0)
        _load(k + 3, xb0, ls0)

        _wait_load(xb2, ls2)
        _compute(xb2)
        _scat(k + 2, xb2, ss2)
        _wait_scat(xb1, ss1)
        _load(k + 4, xb1, ls1)
        return 0
    lax.fori_loop(1, (NCHUNK - 2) // 3, _triple, 0)

    _wait_load(xb0, ls0)
    _compute(xb0)
    _scat(NCHUNK - 2, xb0, ss0)
    _wait_scat(xb2, ss2)

    _wait_load(xb1, ls1)
    _compute(xb1)
    _scat(NCHUNK - 1, xb1, ss1)
    _wait_scat(xb0, ss0)
    _wait_scat(xb1, ss1)

    plsc.subcore_barrier()
    pltpu.sync_copy(acc.at[pl.ds(s * SEG_PER_TILE, SEG_PER_TILE)],
                    out_hbm.at[pl.ds(c * S + s * SEG_PER_TILE, SEG_PER_TILE)])


@jax.jit
def _weight_and_sum(x, idx3, w_flat, b16):
    mesh = plsc.VectorSubcoreMesh(core_axis_name="c", subcore_axis_name="s",
                                  num_cores=NC, num_subcores=NS)
    f = pl.kernel(
        _body,
        out_type=jax.ShapeDtypeStruct((NC * S, D), jnp.float32),
        mesh=mesh,
        scratch_types=[
            pltpu.VMEM((C, D), jnp.float32),
            pltpu.VMEM((C, D), jnp.float32),
            pltpu.VMEM((C, D), jnp.float32),
            pltpu.VMEM((NCHUNK, C), jnp.int32),
            pltpu.VMEM((D,), jnp.float32),
            pltpu.VMEM((L,), jnp.float32),
            pltpu.VMEM((SEG_PER_TILE, D), jnp.float32),
            pltpu.VMEM_SHARED((S, D), jnp.float32),
            pltpu.SemaphoreType.DMA,
            pltpu.SemaphoreType.DMA,
            pltpu.SemaphoreType.DMA,
            pltpu.SemaphoreType.DMA,
            pltpu.SemaphoreType.DMA,
            pltpu.SemaphoreType.DMA,
        ],
        compiler_params=pltpu.CompilerParams(use_tc_tiling_on_sc=False,
                                             needs_layout_passes=False),
    )
    partials = f(x, idx3, w_flat, b16)
    return partials[:S] + partials[S:]


def kernel(x, batch, W, b):
    idx3 = batch.reshape(NW, NCHUNK, C)
    w_flat = W.reshape(D)
    b16 = jnp.broadcast_to(b, (L,))
    return _weight_and_sum(x, idx3, w_flat, b16)


# speedup vs baseline: 2.7430x; 1.1604x over previous
import jax
import jax.numpy as jnp
from jax import lax
from jax.experimental import pallas as pl
from jax.experimental.pallas import tpu as pltpu
from jax.experimental.pallas import tpu_sc as plsc

N_NODES = 100000
D = 512
S = 1024
NC = 2
NS = 16
NW = NC * NS
RPT = N_NODES // NW
C = 25
NCHUNK = RPT // C
L = 16
DV = D // L
SEG_PER_TILE = S // NS


def _body(x_hbm, idx_hbm, w_hbm, b_hbm, out_hbm, xb0, xb1, xb2, idxbuf,
          wbuf, bbuf, zbuf, acc, ls0, ls1, ls2, ss0, ss1, ss2):
    c = lax.axis_index("c")
    s = lax.axis_index("s")
    wid = c * NS + s
    row0 = wid * RPT

    pltpu.sync_copy(w_hbm, wbuf)
    pltpu.sync_copy(b_hbm, bbuf)
    pltpu.sync_copy(idx_hbm.at[wid], idxbuf)

    def _zero_row(r, _):
        for j in range(DV):
            zbuf[r, pl.ds(L * j, L)] = jnp.zeros((L,), jnp.float32)
        return 0
    lax.fori_loop(0, SEG_PER_TILE, _zero_row, 0)
    pltpu.sync_copy(zbuf, acc.at[pl.ds(s * SEG_PER_TILE, SEG_PER_TILE)])
    plsc.subcore_barrier()

    bias = bbuf[:]
    ws = [wbuf[pl.ds(L * j, L)] for j in range(DV)]

    def _load(k, xb, sem):
        pltpu.async_copy(x_hbm.at[pl.ds(row0 + k * C, C)], xb, sem)

    def _wait_load(xb, sem):
        pltpu.make_async_copy(x_hbm.at[pl.ds(row0, C)], xb, sem).wait()

    def _scat(k, xb, sem):
        pltpu.async_copy(xb, acc.at[idxbuf.at[k]], sem, add=True)

    def _wait_scat(xb, sem):
        pltpu.make_async_copy(xb, acc.at[idxbuf.at[0]], sem).wait()

    def _do_row(xb, r):
        xs = [xb[r, pl.ds(L * j, L)] for j in range(DV)]
        accs = [xs[j] * ws[j] for j in range(4)]
        for j in range(4, DV):
            accs[j % 4] = accs[j % 4] + xs[j] * ws[j]
        accv = (accs[0] + accs[1]) + (accs[2] + accs[3])
        dot = jnp.sum(accv)
        z = jnp.full((L,), dot, jnp.float32) + bias
        wv = 1.0 / (1.0 + jnp.exp(-z))
        for j in range(DV):
            xb[r, pl.ds(L * j, L)] = xs[j] * wv

    def _compute(xb):
        def _row_group(r, _):
            for u in range(5):
                _do_row(xb, 5 * r + u)
            return 0
        lax.fori_loop(0, C // 5, _row_group, 0)

    _load(0, xb0, ls0)
    _load(1, xb1, ls1)

    _wait_load(xb0, ls0)
    _compute(xb0)
    _scat(0, xb0, ss0)
    _load(2, xb2, ls2)

    _wait_load(xb1, ls1)
    _compute(xb1)
    _scat(1, xb1, ss1)
    _wait_scat(xb0, ss0)
    _load(3, xb0, ls0)

    _wait_load(xb2, ls2)
    _compute(xb2)
    _scat(2, xb2, ss2)
    _wait_scat(xb1, ss1)
    _load(4, xb1, ls1)

    def _triple(i, _):
        k = 3 * i
        _wait_load(xb0, ls0)
        _compute(xb0)
        _scat(k, xb0, ss0)
        _wait_scat(xb2, ss2)
        _load(k + 2, xb2, ls2)

        _wait_load(xb1, ls1)
        _compute(xb1)
        _scat(k + 1, xb1, ss1)
        _wait_scat(xb0, ss0)
        _load(k + 3, xb0, ls0)

        _wait_load(xb2, ls2)
        _compute(xb2)
        _scat(k + 2, xb2, ss2)
        _wait_scat(xb1, ss1)
        _load(k + 4, xb1, ls1)
        return 0
    lax.fori_loop(1, (NCHUNK - 2) // 3, _triple, 0)

    _wait_load(xb0, ls0)
    _compute(xb0)
    _scat(NCHUNK - 2, xb0, ss0)
    _wait_scat(xb2, ss2)

    _wait_load(xb1, ls1)
    _compute(xb1)
    _scat(NCHUNK - 1, xb1, ss1)
    _wait_scat(xb0, ss0)
    _wait_scat(xb1, ss1)

    plsc.subcore_barrier()
    pltpu.sync_copy(acc.at[pl.ds(s * SEG_PER_TILE, SEG_PER_TILE)],
                    out_hbm.at[pl.ds(c * S + s * SEG_PER_TILE, SEG_PER_TILE)])


@jax.jit
def _weight_and_sum(x, idx3, w_flat, b16):
    mesh = plsc.VectorSubcoreMesh(core_axis_name="c", subcore_axis_name="s",
                                  num_cores=NC, num_subcores=NS)
    f = pl.kernel(
        _body,
        out_type=jax.ShapeDtypeStruct((NC * S, D), jnp.float32),
        mesh=mesh,
        scratch_types=[
            pltpu.VMEM((C, D), jnp.float32),
            pltpu.VMEM((C, D), jnp.float32),
            pltpu.VMEM((C, D), jnp.float32),
            pltpu.VMEM((NCHUNK, C), jnp.int32),
            pltpu.VMEM((D,), jnp.float32),
            pltpu.VMEM((L,), jnp.float32),
            pltpu.VMEM((SEG_PER_TILE, D), jnp.float32),
            pltpu.VMEM_SHARED((S, D), jnp.float32),
            pltpu.SemaphoreType.DMA,
            pltpu.SemaphoreType.DMA,
            pltpu.SemaphoreType.DMA,
            pltpu.SemaphoreType.DMA,
            pltpu.SemaphoreType.DMA,
            pltpu.SemaphoreType.DMA,
        ],
        compiler_params=pltpu.CompilerParams(use_tc_tiling_on_sc=False,
                                             needs_layout_passes=False),
    )
    partials = f(x, idx3, w_flat, b16)
    return partials[:S] + partials[S:]


def kernel(x, batch, W, b):
    idx3 = batch.reshape(NW, NCHUNK, C)
    w_flat = W.reshape(D)
    b16 = jnp.broadcast_to(b, (L,))
    return _weight_and_sum(x, idx3, w_flat, b16)
